# baseline (speedup 1.0000x reference)
import functools

import jax
import jax.numpy as jnp
from jax import lax
from jax.experimental import pallas as pl
from jax.experimental.pallas import tpu as pltpu
from jax.experimental.pallas import tpu_sc as plsc

N = 50000
E = 800000
NC = 2
NS = 16


DEG_W = 8
DEG_CHUNK = 5000
DEG_PER_TILE = E // (NC * NS)

_sc_mesh = plsc.VectorSubcoreMesh(
    core_axis_name="c", subcore_axis_name="s", num_cores=NC, num_subcores=NS)
_sc_params = pltpu.CompilerParams(use_tc_tiling_on_sc=False)

ROWS_A = 3128
ROWS_B = N - (NS - 1) * ROWS_A


def _sliced_copy(src_ref, dst_ref, s):
    @pl.when(s < NS - 1)
    def _():
        pltpu.sync_copy(src_ref.at[pl.ds(s * ROWS_A, ROWS_A)],
                        dst_ref.at[pl.ds(s * ROWS_A, ROWS_A)])

    @pl.when(s == NS - 1)
    def _():
        pltpu.sync_copy(src_ref.at[pl.ds((NS - 1) * ROWS_A, ROWS_B)],
                        dst_ref.at[pl.ds((NS - 1) * ROWS_A, ROWS_B)])


ZCHUNK = 1000


def _zero_fill(zsrc, acc_ref, s):
    def fill(row0, nrows):
        full = nrows // ZCHUNK
        for i in range(full):
            pltpu.sync_copy(zsrc, acc_ref.at[pl.ds(row0 + i * ZCHUNK, ZCHUNK)])
        rem = nrows - full * ZCHUNK
        if rem:
            pltpu.sync_copy(zsrc.at[pl.ds(0, rem)],
                            acc_ref.at[pl.ds(row0 + full * ZCHUNK, rem)])

    @pl.when(s < NS - 1)
    def _():
        fill(s * ROWS_A, ROWS_A)

    @pl.when(s == NS - 1)
    def _():
        fill((NS - 1) * ROWS_A, ROWS_B)


@functools.partial(
    pl.kernel,
    out_type=(
        jax.ShapeDtypeStruct((N, DEG_W), jnp.float32),
        jax.ShapeDtypeStruct((N, DEG_W), jnp.float32),
    ),
    mesh=_sc_mesh,
    compiler_params=_sc_params,
    scratch_types=[
        pltpu.VMEM((DEG_CHUNK,), jnp.int32),
        pltpu.VMEM((DEG_CHUNK, DEG_W), jnp.float32),
        pltpu.VMEM_SHARED((N, DEG_W), jnp.float32),
    ],
)
def _deg_sc(dst_hbm, zeros8_hbm, ones8_hbm, out0_hbm, out1_hbm,
            didx, ones_v, acc):
    c = lax.axis_index("c")
    s = lax.axis_index("s")

    pltpu.sync_copy(ones8_hbm, ones_v)
    _zero_fill(zeros8_hbm, acc, s)
    plsc.subcore_barrier()

    base = (c * NS + s) * DEG_PER_TILE

    def body(k, carry):
        off = base + k * DEG_CHUNK
        pltpu.sync_copy(dst_hbm.at[pl.ds(off, DEG_CHUNK)], didx)
        pltpu.sync_copy(ones_v, acc.at[didx], add=True)
        return carry

    lax.fori_loop(0, DEG_PER_TILE // DEG_CHUNK, body, 0)
    plsc.subcore_barrier()

    @pl.when(c == 0)
    def _():
        _sliced_copy(acc, out0_hbm, s)

    @pl.when(c == 1)
    def _():
        _sliced_copy(acc, out1_hbm, s)


SEG_CHUNK = 200
SEG_CPB = 25
SEG_NBLK = 10
SEG_IDX_ROWS = E // SEG_CHUNK
HALF = 32


@functools.partial(
    pl.kernel,
    out_type=(
        jax.ShapeDtypeStruct((N, HALF), jnp.float32),
        jax.ShapeDtypeStruct((N, HALF), jnp.float32),
    ),
    mesh=_sc_mesh,
    compiler_params=_sc_params,
    scratch_types=[
        pltpu.VMEM((SEG_CPB, SEG_CHUNK), jnp.int32),
        pltpu.VMEM((SEG_CPB, SEG_CHUNK), jnp.int32),
        pltpu.VMEM((SEG_CHUNK, HALF), jnp.float32),
        pltpu.VMEM((SEG_CHUNK, HALF), jnp.float32),
        pltpu.VMEM_SHARED((N, HALF), jnp.float32),
        pltpu.SemaphoreType.DMA,
        pltpu.SemaphoreType.DMA,
        pltpu.SemaphoreType.DMA,
        pltpu.SemaphoreType.DMA,
    ],
)
def _seg_sc(g0_hbm, g1_hbm, src2d_hbm, dst2d_hbm, zeros32_hbm,
            out0_hbm, out1_hbm,
            sblk, dblk, rows0, rows1, acc, gsem0, gsem1, ssem0, ssem1):
    c = lax.axis_index("c")
    s = lax.axis_index("s")

    _zero_fill(zeros32_hbm, acc, s)
    plsc.subcore_barrier()

    rowsb = (rows0, rows1)
    gsem = (gsem0, gsem1)
    ssem = (ssem0, ssem1)

    def make_body(g_ref):
        def body(blk, carry):
            base = s * (SEG_NBLK * SEG_CPB) + blk * SEG_CPB
            pltpu.sync_copy(src2d_hbm.at[pl.ds(base, SEG_CPB)], sblk)
            pltpu.sync_copy(dst2d_hbm.at[pl.ds(base, SEG_CPB)], dblk)
            g_descs = [None] * SEG_CPB
            s_descs = [None] * SEG_CPB
            for j in range(SEG_CPB):
                b = j & 1
                if j >= 2:
                    s_descs[j - 2].wait()
                g_descs[j] = pltpu.async_copy(
                    g_ref.at[sblk.at[j]], rowsb[b], gsem[b])
                if j >= 1:
                    g_descs[j - 1].wait()
                    s_descs[j - 1] = pltpu.async_copy(
                        rowsb[1 - b], acc.at[dblk.at[j - 1]], ssem[1 - b],
                        add=True)
            last = SEG_CPB - 1
            g_descs[last].wait()
            s_descs[last] = pltpu.async_copy(
                rowsb[last & 1], acc.at[dblk.at[last]], ssem[last & 1],
                add=True)
            s_descs[last - 1].wait()
            s_descs[last].wait()
            return carry
        return body

    @pl.when(c == 0)
    def _():
        lax.fori_loop(0, SEG_NBLK, make_body(g0_hbm), 0)

    @pl.when(c == 1)
    def _():
        lax.fori_loop(0, SEG_NBLK, make_body(g1_hbm), 0)

    plsc.subcore_barrier()

    @pl.when(c == 0)
    def _():
        _sliced_copy(acc, out0_hbm, s)

    @pl.when(c == 1)
    def _():
        _sliced_copy(acc, out1_hbm, s)


BLK = 1000
GRID = N // BLK
MBLK = 5000
MGRID = N // MBLK


def _leaky(x):
    return jnp.where(x >= 0, x, 0.01 * x)


def _row_spec(w):
    return pl.BlockSpec((BLK, w), lambda i: (i, 0))


def _mrow_spec(w):
    return pl.BlockSpec((MBLK, w), lambda i: (i, 0))


def _full_spec(a, b):
    return pl.BlockSpec((a, b), lambda i: (0, 0))


def _tc_encode_body(screen, des, tweet, profile, personal, deg0, deg1,
                    Ws, bs, Wd, bd, Wt, bt, Wp, bp, Wq, bq, Wl, bl, Wg, bg,
                    x1_o, dinv_o, g0_o, g1_o, selfw_o):
    s = _leaky(jnp.dot(screen[...], Ws[...],
                       preferred_element_type=jnp.float32) + bs[...])
    d = _leaky(jnp.dot(des[...], Wd[...],
                       preferred_element_type=jnp.float32) + bd[...])
    t = _leaky(jnp.dot(tweet[...], Wt[...],
                       preferred_element_type=jnp.float32) + bt[...])
    p = _leaky(jnp.dot(profile[...], Wp[...],
                       preferred_element_type=jnp.float32) + bp[...])
    q = _leaky(jnp.dot(personal[...], Wq[...],
                       preferred_element_type=jnp.float32) + bq[...])
    x1 = jnp.concatenate([s, d, t, p, q], axis=1)
    x = _leaky(jnp.dot(x1, Wl[...], preferred_element_type=jnp.float32)
               + bl[...])
    deg = deg0[:, 0:1] + deg1[:, 0:1] + 1.0
    dinv = lax.rsqrt(deg)
    h = jnp.dot(x, Wg[...], preferred_element_type=jnp.float32)
    g = h * dinv
    x1_o[...] = x1
    dinv_o[...] = jnp.broadcast_to(dinv, (BLK, 8))
    g0_o[...] = g[:, :HALF]
    g1_o[...] = g[:, HALF:]
    selfw_o[...] = g * dinv + bg[...] + x1


_tc_encode = pl.pallas_call(
    _tc_encode_body,
    grid=(GRID,),
    in_specs=[
        _row_spec(768), _row_spec(768), _row_spec(768),
        _row_spec(5), _row_spec(7), _row_spec(8), _row_spec(8),
        _full_spec(768, 16), _full_spec(1, 16),
        _full_spec(768, 16), _full_spec(1, 16),
        _full_spec(768, 16), _full_spec(1, 16),
        _full_spec(5, 8), _full_spec(1, 8),
        _full_spec(7, 8), _full_spec(1, 8),
        _full_spec(64, 64), _full_spec(1, 64),
        _full_spec(64, 64), _full_spec(1, 64),
    ],
    out_specs=[
        _row_spec(64), _row_spec(8), _row_spec(HALF), _row_spec(HALF),
        _row_spec(64),
    ],
    out_shape=[
        jax.ShapeDtypeStruct((N, 64), jnp.float32),
        jax.ShapeDtypeStruct((N, 8), jnp.float32),
        jax.ShapeDtypeStruct((N, HALF), jnp.float32),
        jax.ShapeDtypeStruct((N, HALF), jnp.float32),
        jax.ShapeDtypeStruct((N, 64), jnp.float32),
    ],
)


def _tc_mid_body(a0, a1, dinv, selfw, x1, W, b,
                 g0_o, g1_o, selfw_o):
    di = dinv[:, 0:1]
    xl = di * jnp.concatenate([a0[...], a1[...]], axis=1) + selfw[...]
    h = jnp.dot(xl, W[...], preferred_element_type=jnp.float32)
    g = h * di
    g0_o[...] = g[:, :HALF]
    g1_o[...] = g[:, HALF:]
    selfw_o[...] = g * di + b[...] + x1[...]


_tc_mid = pl.pallas_call(
    _tc_mid_body,
    grid=(MGRID,),
    in_specs=[
        _mrow_spec(HALF), _mrow_spec(HALF), _mrow_spec(8), _mrow_spec(64),
        _mrow_spec(64),
        _full_spec(64, 64), _full_spec(1, 64),
    ],
    out_specs=[_mrow_spec(HALF), _mrow_spec(HALF), _mrow_spec(64)],
    out_shape=[
        jax.ShapeDtypeStruct((N, HALF), jnp.float32),
        jax.ShapeDtypeStruct((N, HALF), jnp.float32),
        jax.ShapeDtypeStruct((N, 64), jnp.float32),
    ],
)


def _tc_final_body(a0, a1, dinv, selfw, Wo1, bo1, Wo2, bo2, y_o):
    di = dinv[:, 0:1]
    x4 = di * jnp.concatenate([a0[...], a1[...]], axis=1) + selfw[...]
    h = _leaky(jnp.dot(x4, Wo1[...], preferred_element_type=jnp.float32)
               + bo1[...])
    z = jnp.dot(h, Wo2[...], preferred_element_type=jnp.float32) + bo2[...]
    y_o[...] = 1.0 / (1.0 + jnp.exp(-z))


_tc_final = pl.pallas_call(
    _tc_final_body,
    grid=(MGRID,),
    in_specs=[
        _mrow_spec(HALF), _mrow_spec(HALF), _mrow_spec(8), _mrow_spec(64),
        _full_spec(64, 64), _full_spec(1, 64),
        _full_spec(64, 2), _full_spec(1, 2),
    ],
    out_specs=[_mrow_spec(2)],
    out_shape=[jax.ShapeDtypeStruct((N, 2), jnp.float32)],
)


def kernel(screen, des, tweet, profile, personal, edge, edgeRelation,
           Ws_w, Ws_b, Wd_w, Wd_b, Wt_w, Wt_b, Wp_w, Wp_b, Wq_w, Wq_b,
           Wl_w, Wl_b, Wg1, bg1, Wg2, bg2, Wg3, bg3, Wo1, bo1, Wo2, bo2):
    src = edge[0].astype(jnp.int32)
    dst = edge[1].astype(jnp.int32)
    src2d = src.reshape(SEG_IDX_ROWS, SEG_CHUNK)
    dst2d = dst.reshape(SEG_IDX_ROWS, SEG_CHUNK)

    zeros8 = jnp.zeros((ZCHUNK, DEG_W), jnp.float32)
    ones8 = jnp.ones((DEG_CHUNK, DEG_W), jnp.float32)
    zeros32 = jnp.zeros((ZCHUNK, HALF), jnp.float32)

    r2 = lambda v: v.reshape(1, -1)

    deg0, deg1 = _deg_sc(dst, zeros8, ones8)

    x1, dinv, g0, g1, selfw = _tc_encode(
        screen, des, tweet, profile, personal, deg0, deg1,
        Ws_w, r2(Ws_b), Wd_w, r2(Wd_b), Wt_w, r2(Wt_b),
        Wp_w, r2(Wp_b), Wq_w, r2(Wq_b), Wl_w, r2(Wl_b), Wg1, r2(bg1))

    for W, b in ((Wg2, bg2), (Wg3, bg3)):
        a0, a1 = _seg_sc(g0, g1, src2d, dst2d, zeros32)
        g0, g1, selfw = _tc_mid(a0, a1, dinv, selfw, x1, W, r2(b))

    a0, a1 = _seg_sc(g0, g1, src2d, dst2d, zeros32)
    y, = _tc_final(a0, a1, dinv, selfw, Wo1, r2(bo1), Wo2, r2(bo2))
    return y


# speedup vs baseline: 1.0198x; 1.0198x over previous
import functools

import jax
import jax.numpy as jnp
from jax import lax
from jax.experimental import pallas as pl
from jax.experimental.pallas import tpu as pltpu
from jax.experimental.pallas import tpu_sc as plsc

N = 50000
E = 800000
NC = 2
NS = 16


DEG_W = 8
DEG_CHUNK = 5000
DEG_PER_TILE = E // (NC * NS)

_sc_mesh = plsc.VectorSubcoreMesh(
    core_axis_name="c", subcore_axis_name="s", num_cores=NC, num_subcores=NS)
_sc_params = pltpu.CompilerParams(use_tc_tiling_on_sc=False)

ROWS_A = 3128
ROWS_B = N - (NS - 1) * ROWS_A


def _sliced_copy(src_ref, dst_ref, s):
    @pl.when(s < NS - 1)
    def _():
        pltpu.sync_copy(src_ref.at[pl.ds(s * ROWS_A, ROWS_A)],
                        dst_ref.at[pl.ds(s * ROWS_A, ROWS_A)])

    @pl.when(s == NS - 1)
    def _():
        pltpu.sync_copy(src_ref.at[pl.ds((NS - 1) * ROWS_A, ROWS_B)],
                        dst_ref.at[pl.ds((NS - 1) * ROWS_A, ROWS_B)])


ZCHUNK = 1000


def _zero_fill(zsrc, acc_ref, s):
    def fill(row0, nrows):
        full = nrows // ZCHUNK
        for i in range(full):
            pltpu.sync_copy(zsrc, acc_ref.at[pl.ds(row0 + i * ZCHUNK, ZCHUNK)])
        rem = nrows - full * ZCHUNK
        if rem:
            pltpu.sync_copy(zsrc.at[pl.ds(0, rem)],
                            acc_ref.at[pl.ds(row0 + full * ZCHUNK, rem)])

    @pl.when(s < NS - 1)
    def _():
        fill(s * ROWS_A, ROWS_A)

    @pl.when(s == NS - 1)
    def _():
        fill((NS - 1) * ROWS_A, ROWS_B)


@functools.partial(
    pl.kernel,
    out_type=(
        jax.ShapeDtypeStruct((N, DEG_W), jnp.float32),
        jax.ShapeDtypeStruct((N, DEG_W), jnp.float32),
    ),
    mesh=_sc_mesh,
    compiler_params=_sc_params,
    scratch_types=[
        pltpu.VMEM((DEG_CHUNK,), jnp.int32),
        pltpu.VMEM((DEG_CHUNK, DEG_W), jnp.float32),
        pltpu.VMEM_SHARED((N, DEG_W), jnp.float32),
    ],
)
def _deg_sc(dst_hbm, zeros8_hbm, ones8_hbm, out0_hbm, out1_hbm,
            didx, ones_v, acc):
    c = lax.axis_index("c")
    s = lax.axis_index("s")

    pltpu.sync_copy(ones8_hbm, ones_v)
    _sliced_copy(zeros8_hbm, acc, s)
    plsc.subcore_barrier()

    base = (c * NS + s) * DEG_PER_TILE

    def body(k, carry):
        off = base + k * DEG_CHUNK
        pltpu.sync_copy(dst_hbm.at[pl.ds(off, DEG_CHUNK)], didx)
        pltpu.sync_copy(ones_v, acc.at[didx], add=True)
        return carry

    lax.fori_loop(0, DEG_PER_TILE // DEG_CHUNK, body, 0)
    plsc.subcore_barrier()

    @pl.when(c == 0)
    def _():
        _sliced_copy(acc, out0_hbm, s)

    @pl.when(c == 1)
    def _():
        _sliced_copy(acc, out1_hbm, s)


SEG_CHUNK = 200
SEG_CPB = 25
SEG_NBLK = 10
SEG_IDX_ROWS = E // SEG_CHUNK
HALF = 32


@functools.partial(
    pl.kernel,
    out_type=(
        jax.ShapeDtypeStruct((N, HALF), jnp.float32),
        jax.ShapeDtypeStruct((N, HALF), jnp.float32),
    ),
    mesh=_sc_mesh,
    compiler_params=_sc_params,
    scratch_types=[
        pltpu.VMEM((SEG_CPB, SEG_CHUNK), jnp.int32),
        pltpu.VMEM((SEG_CPB, SEG_CHUNK), jnp.int32),
        pltpu.VMEM((SEG_CHUNK, HALF), jnp.float32),
        pltpu.VMEM((SEG_CHUNK, HALF), jnp.float32),
        pltpu.VMEM_SHARED((N, HALF), jnp.float32),
        pltpu.SemaphoreType.DMA,
        pltpu.SemaphoreType.DMA,
        pltpu.SemaphoreType.DMA,
        pltpu.SemaphoreType.DMA,
    ],
)
def _seg_sc(g0_hbm, g1_hbm, src2d_hbm, dst2d_hbm, zeros32_hbm,
            out0_hbm, out1_hbm,
            sblk, dblk, rows0, rows1, acc, gsem0, gsem1, ssem0, ssem1):
    c = lax.axis_index("c")
    s = lax.axis_index("s")

    _sliced_copy(zeros32_hbm, acc, s)
    plsc.subcore_barrier()

    rowsb = (rows0, rows1)
    gsem = (gsem0, gsem1)
    ssem = (ssem0, ssem1)

    def make_body(g_ref):
        def body(blk, carry):
            base = s * (SEG_NBLK * SEG_CPB) + blk * SEG_CPB
            pltpu.sync_copy(src2d_hbm.at[pl.ds(base, SEG_CPB)], sblk)
            pltpu.sync_copy(dst2d_hbm.at[pl.ds(base, SEG_CPB)], dblk)
            g_descs = [None] * SEG_CPB
            s_descs = [None] * SEG_CPB
            for j in range(SEG_CPB):
                b = j & 1
                if j >= 2:
                    s_descs[j - 2].wait()
                g_descs[j] = pltpu.async_copy(
                    g_ref.at[sblk.at[j]], rowsb[b], gsem[b])
                if j >= 1:
                    g_descs[j - 1].wait()
                    s_descs[j - 1] = pltpu.async_copy(
                        rowsb[1 - b], acc.at[dblk.at[j - 1]], ssem[1 - b],
                        add=True)
            last = SEG_CPB - 1
            g_descs[last].wait()
            s_descs[last] = pltpu.async_copy(
                rowsb[last & 1], acc.at[dblk.at[last]], ssem[last & 1],
                add=True)
            s_descs[last - 1].wait()
            s_descs[last].wait()
            return carry
        return body

    @pl.when(c == 0)
    def _():
        lax.fori_loop(0, SEG_NBLK, make_body(g0_hbm), 0)

    @pl.when(c == 1)
    def _():
        lax.fori_loop(0, SEG_NBLK, make_body(g1_hbm), 0)

    plsc.subcore_barrier()

    @pl.when(c == 0)
    def _():
        _sliced_copy(acc, out0_hbm, s)

    @pl.when(c == 1)
    def _():
        _sliced_copy(acc, out1_hbm, s)


BLK = 1000
GRID = N // BLK
MBLK = 5000
MGRID = N // MBLK


def _leaky(x):
    return jnp.where(x >= 0, x, 0.01 * x)


def _row_spec(w):
    return pl.BlockSpec((BLK, w), lambda i: (i, 0))


def _mrow_spec(w):
    return pl.BlockSpec((MBLK, w), lambda i: (i, 0))


def _full_spec(a, b):
    return pl.BlockSpec((a, b), lambda i: (0, 0))


def _tc_encode_body(screen, des, tweet, profile, personal, deg0, deg1,
                    Ws, bs, Wd, bd, Wt, bt, Wp, bp, Wq, bq, Wl, bl, Wg, bg,
                    x1_o, dinv_o, g0_o, g1_o, selfw_o):
    s = _leaky(jnp.dot(screen[...], Ws[...],
                       preferred_element_type=jnp.float32) + bs[...])
    d = _leaky(jnp.dot(des[...], Wd[...],
                       preferred_element_type=jnp.float32) + bd[...])
    t = _leaky(jnp.dot(tweet[...], Wt[...],
                       preferred_element_type=jnp.float32) + bt[...])
    p = _leaky(jnp.dot(profile[...], Wp[...],
                       preferred_element_type=jnp.float32) + bp[...])
    q = _leaky(jnp.dot(personal[...], Wq[...],
                       preferred_element_type=jnp.float32) + bq[...])
    x1 = jnp.concatenate([s, d, t, p, q], axis=1)
    x = _leaky(jnp.dot(x1, Wl[...], preferred_element_type=jnp.float32)
               + bl[...])
    deg = deg0[:, 0:1] + deg1[:, 0:1] + 1.0
    dinv = lax.rsqrt(deg)
    h = jnp.dot(x, Wg[...], preferred_element_type=jnp.float32)
    g = h * dinv
    x1_o[...] = x1
    dinv_o[...] = jnp.broadcast_to(dinv, (BLK, 8))
    g0_o[...] = g[:, :HALF]
    g1_o[...] = g[:, HALF:]
    selfw_o[...] = g * dinv + bg[...] + x1


_tc_encode = pl.pallas_call(
    _tc_encode_body,
    grid=(GRID,),
    in_specs=[
        _row_spec(768), _row_spec(768), _row_spec(768),
        _row_spec(5), _row_spec(7), _row_spec(8), _row_spec(8),
        _full_spec(768, 16), _full_spec(1, 16),
        _full_spec(768, 16), _full_spec(1, 16),
        _full_spec(768, 16), _full_spec(1, 16),
        _full_spec(5, 8), _full_spec(1, 8),
        _full_spec(7, 8), _full_spec(1, 8),
        _full_spec(64, 64), _full_spec(1, 64),
        _full_spec(64, 64), _full_spec(1, 64),
    ],
    out_specs=[
        _row_spec(64), _row_spec(8), _row_spec(HALF), _row_spec(HALF),
        _row_spec(64),
    ],
    out_shape=[
        jax.ShapeDtypeStruct((N, 64), jnp.float32),
        jax.ShapeDtypeStruct((N, 8), jnp.float32),
        jax.ShapeDtypeStruct((N, HALF), jnp.float32),
        jax.ShapeDtypeStruct((N, HALF), jnp.float32),
        jax.ShapeDtypeStruct((N, 64), jnp.float32),
    ],
)


def _tc_mid_body(a0, a1, dinv, selfw, x1, W, b,
                 g0_o, g1_o, selfw_o):
    di = dinv[:, 0:1]
    xl = di * jnp.concatenate([a0[...], a1[...]], axis=1) + selfw[...]
    h = jnp.dot(xl, W[...], preferred_element_type=jnp.float32)
    g = h * di
    g0_o[...] = g[:, :HALF]
    g1_o[...] = g[:, HALF:]
    selfw_o[...] = g * di + b[...] + x1[...]


_tc_mid = pl.pallas_call(
    _tc_mid_body,
    grid=(MGRID,),
    in_specs=[
        _mrow_spec(HALF), _mrow_spec(HALF), _mrow_spec(8), _mrow_spec(64),
        _mrow_spec(64),
        _full_spec(64, 64), _full_spec(1, 64),
    ],
    out_specs=[_mrow_spec(HALF), _mrow_spec(HALF), _mrow_spec(64)],
    out_shape=[
        jax.ShapeDtypeStruct((N, HALF), jnp.float32),
        jax.ShapeDtypeStruct((N, HALF), jnp.float32),
        jax.ShapeDtypeStruct((N, 64), jnp.float32),
    ],
)


def _tc_final_body(a0, a1, dinv, selfw, Wo1, bo1, Wo2, bo2, y_o):
    di = dinv[:, 0:1]
    x4 = di * jnp.concatenate([a0[...], a1[...]], axis=1) + selfw[...]
    h = _leaky(jnp.dot(x4, Wo1[...], preferred_element_type=jnp.float32)
               + bo1[...])
    z = jnp.dot(h, Wo2[...], preferred_element_type=jnp.float32) + bo2[...]
    y_o[...] = 1.0 / (1.0 + jnp.exp(-z))


_tc_final = pl.pallas_call(
    _tc_final_body,
    grid=(MGRID,),
    in_specs=[
        _mrow_spec(HALF), _mrow_spec(HALF), _mrow_spec(8), _mrow_spec(64),
        _full_spec(64, 64), _full_spec(1, 64),
        _full_spec(64, 2), _full_spec(1, 2),
    ],
    out_specs=[_mrow_spec(2)],
    out_shape=[jax.ShapeDtypeStruct((N, 2), jnp.float32)],
)


def kernel(screen, des, tweet, profile, personal, edge, edgeRelation,
           Ws_w, Ws_b, Wd_w, Wd_b, Wt_w, Wt_b, Wp_w, Wp_b, Wq_w, Wq_b,
           Wl_w, Wl_b, Wg1, bg1, Wg2, bg2, Wg3, bg3, Wo1, bo1, Wo2, bo2):
    src = edge[0].astype(jnp.int32)
    dst = edge[1].astype(jnp.int32)
    src2d = src.reshape(SEG_IDX_ROWS, SEG_CHUNK)
    dst2d = dst.reshape(SEG_IDX_ROWS, SEG_CHUNK)

    zeros8 = jnp.zeros((N, DEG_W), jnp.float32)
    ones8 = jnp.ones((DEG_CHUNK, DEG_W), jnp.float32)
    zeros32 = jnp.zeros((N, HALF), jnp.float32)

    r2 = lambda v: v.reshape(1, -1)

    deg0, deg1 = _deg_sc(dst, zeros8, ones8)

    x1, dinv, g0, g1, selfw = _tc_encode(
        screen, des, tweet, profile, personal, deg0, deg1,
        Ws_w, r2(Ws_b), Wd_w, r2(Wd_b), Wt_w, r2(Wt_b),
        Wp_w, r2(Wp_b), Wq_w, r2(Wq_b), Wl_w, r2(Wl_b), Wg1, r2(bg1))

    for W, b in ((Wg2, bg2), (Wg3, bg3)):
        a0, a1 = _seg_sc(g0, g1, src2d, dst2d, zeros32)
        g0, g1, selfw = _tc_mid(a0, a1, dinv, selfw, x1, W, r2(b))

    a0, a1 = _seg_sc(g0, g1, src2d, dst2d, zeros32)
    y, = _tc_final(a0, a1, dinv, selfw, Wo1, r2(bo1), Wo2, r2(bo2))
    return y


# speedup vs baseline: 1.0580x; 1.0375x over previous
import functools

import jax
import jax.numpy as jnp
from jax import lax
from jax.experimental import pallas as pl
from jax.experimental.pallas import tpu as pltpu
from jax.experimental.pallas import tpu_sc as plsc

N = 50000
E = 800000
NC = 2
NS = 16


DEG_W = 8
DEG_CHUNK = 5000
DEG_PER_TILE = E // (NC * NS)

_sc_mesh = plsc.VectorSubcoreMesh(
    core_axis_name="c", subcore_axis_name="s", num_cores=NC, num_subcores=NS)
_sc_params = pltpu.CompilerParams(use_tc_tiling_on_sc=False)

ROWS_A = 3128
ROWS_B = N - (NS - 1) * ROWS_A


def _sliced_copy(src_ref, dst_ref, s):
    @pl.when(s < NS - 1)
    def _():
        pltpu.sync_copy(src_ref.at[pl.ds(s * ROWS_A, ROWS_A)],
                        dst_ref.at[pl.ds(s * ROWS_A, ROWS_A)])

    @pl.when(s == NS - 1)
    def _():
        pltpu.sync_copy(src_ref.at[pl.ds((NS - 1) * ROWS_A, ROWS_B)],
                        dst_ref.at[pl.ds((NS - 1) * ROWS_A, ROWS_B)])


ZCHUNK = 1000


def _zero_fill(zsrc, acc_ref, s):
    def fill(row0, nrows):
        full = nrows // ZCHUNK
        for i in range(full):
            pltpu.sync_copy(zsrc, acc_ref.at[pl.ds(row0 + i * ZCHUNK, ZCHUNK)])
        rem = nrows - full * ZCHUNK
        if rem:
            pltpu.sync_copy(zsrc.at[pl.ds(0, rem)],
                            acc_ref.at[pl.ds(row0 + full * ZCHUNK, rem)])

    @pl.when(s < NS - 1)
    def _():
        fill(s * ROWS_A, ROWS_A)

    @pl.when(s == NS - 1)
    def _():
        fill((NS - 1) * ROWS_A, ROWS_B)


@functools.partial(
    pl.kernel,
    out_type=(
        jax.ShapeDtypeStruct((N, DEG_W), jnp.float32),
        jax.ShapeDtypeStruct((N, DEG_W), jnp.float32),
    ),
    mesh=_sc_mesh,
    compiler_params=_sc_params,
    scratch_types=[
        pltpu.VMEM((DEG_CHUNK,), jnp.int32),
        pltpu.VMEM((DEG_CHUNK, DEG_W), jnp.float32),
        pltpu.VMEM_SHARED((N, DEG_W), jnp.float32),
    ],
)
def _deg_sc(dst_hbm, zeros8_hbm, ones8_hbm, out0_hbm, out1_hbm,
            didx, ones_v, acc):
    c = lax.axis_index("c")
    s = lax.axis_index("s")

    pltpu.sync_copy(ones8_hbm, ones_v)
    _sliced_copy(zeros8_hbm, acc, s)
    plsc.subcore_barrier()

    base = (c * NS + s) * DEG_PER_TILE

    def body(k, carry):
        off = base + k * DEG_CHUNK
        pltpu.sync_copy(dst_hbm.at[pl.ds(off, DEG_CHUNK)], didx)
        pltpu.sync_copy(ones_v, acc.at[didx], add=True)
        return carry

    lax.fori_loop(0, DEG_PER_TILE // DEG_CHUNK, body, 0)
    plsc.subcore_barrier()

    @pl.when(c == 0)
    def _():
        _sliced_copy(acc, out0_hbm, s)

    @pl.when(c == 1)
    def _():
        _sliced_copy(acc, out1_hbm, s)


SEG_CHUNK = 200
SEG_CPB = 10
SEG_NBLK = 25
SEG_IDX_ROWS = E // SEG_CHUNK
HALF = 32


@functools.partial(
    pl.kernel,
    out_type=(
        jax.ShapeDtypeStruct((N, HALF), jnp.float32),
        jax.ShapeDtypeStruct((N, HALF), jnp.float32),
    ),
    mesh=_sc_mesh,
    compiler_params=_sc_params,
    scratch_types=[
        pltpu.VMEM((SEG_CPB, SEG_CHUNK), jnp.int32),
        pltpu.VMEM((SEG_CPB, SEG_CHUNK), jnp.int32),
        pltpu.VMEM((SEG_CPB, SEG_CHUNK), jnp.int32),
        pltpu.VMEM((SEG_CPB, SEG_CHUNK), jnp.int32),
        pltpu.VMEM((SEG_CPB, SEG_CHUNK), jnp.int32),
        pltpu.VMEM((SEG_CPB, SEG_CHUNK), jnp.int32),
        pltpu.VMEM((SEG_CHUNK, HALF), jnp.float32),
        pltpu.VMEM((SEG_CHUNK, HALF), jnp.float32),
        pltpu.VMEM_SHARED((N, HALF), jnp.float32),
        pltpu.SemaphoreType.DMA,
        pltpu.SemaphoreType.DMA,
        pltpu.SemaphoreType.DMA,
        pltpu.SemaphoreType.DMA,
        pltpu.SemaphoreType.DMA,
        pltpu.SemaphoreType.DMA,
        pltpu.SemaphoreType.DMA,
    ],
)
def _seg_sc(g0_hbm, g1_hbm, src2d_hbm, dst2d_hbm, zeros32_hbm,
            out0_hbm, out1_hbm,
            sblk0, dblk0, sblk1, dblk1, sblk2, dblk2, rows0, rows1, acc,
            gsem0, gsem1, ssem0, ssem1, isem0, isem1, isem2):
    c = lax.axis_index("c")
    s = lax.axis_index("s")

    _sliced_copy(zeros32_hbm, acc, s)
    plsc.subcore_barrier()

    rowsb = (rows0, rows1)
    gsem = (gsem0, gsem1)
    ssem = (ssem0, ssem1)
    idxb = ((sblk0, dblk0, isem0), (sblk1, dblk1, isem1),
            (sblk2, dblk2, isem2))
    tile_row0 = s * (SEG_NBLK * SEG_CPB)

    def load_idx(blk, bufs):
        sb, db, sem = bufs
        base = tile_row0 + blk * SEG_CPB
        d1 = pltpu.async_copy(src2d_hbm.at[pl.ds(base, SEG_CPB)], sb, sem)
        d2 = pltpu.async_copy(dst2d_hbm.at[pl.ds(base, SEG_CPB)], db, sem)
        return d1, d2

    def wait_idx(bufs):
        sb, db, sem = bufs
        pltpu.make_async_copy(
            src2d_hbm.at[pl.ds(0, SEG_CPB)], sb, sem).wait()
        pltpu.make_async_copy(
            dst2d_hbm.at[pl.ds(0, SEG_CPB)], db, sem).wait()

    def wait_prev_scatter(b, db):
        pltpu.make_async_copy(rowsb[b], acc.at[db.at[0]], ssem[b]).wait()

    def process_block(g_ref, bufs, first, prefetch):
        sb, db, sem = bufs
        g_descs = [None] * SEG_CPB
        s_descs = [None] * SEG_CPB
        for j in range(SEG_CPB):
            b = j & 1
            if j >= 2:
                s_descs[j - 2].wait()
            elif not first:
                wait_prev_scatter(b, db)
            g_descs[j] = pltpu.async_copy(g_ref.at[sb.at[j]], rowsb[b],
                                          gsem[b])
            if j >= 1:
                g_descs[j - 1].wait()
                s_descs[j - 1] = pltpu.async_copy(
                    rowsb[(j - 1) & 1], acc.at[db.at[j - 1]],
                    ssem[(j - 1) & 1], add=True)
        last = SEG_CPB - 1
        g_descs[last].wait()
        s_descs[last] = pltpu.async_copy(
            rowsb[last & 1], acc.at[db.at[last]], ssem[last & 1], add=True)
        if prefetch is not None:
            prefetch()

    def run(g_ref):
        d0 = load_idx(0, idxb[0])
        load_idx(1, idxb[1])
        load_idx(2, idxb[2])
        d0[0].wait()
        d0[1].wait()
        process_block(g_ref, idxb[0], True, None)

        def body(j2, carry):
            for k, off in ((1, 1), (2, 2), (0, 3)):
                blk = 3 * j2 + off
                bufs = idxb[k]
                wait_idx(bufs)

                def prefetch(blk=blk, k=k):
                    @pl.when(blk + 2 <= SEG_NBLK - 1)
                    def _():
                        load_idx(blk + 2, idxb[(k + 2) % 3])

                process_block(g_ref, bufs, False, prefetch)
            return carry

        lax.fori_loop(0, (SEG_NBLK - 1) // 3, body, 0)
        wait_prev_scatter(0, dblk0)
        wait_prev_scatter(1, dblk0)

    @pl.when(c == 0)
    def _():
        run(g0_hbm)

    @pl.when(c == 1)
    def _():
        run(g1_hbm)

    plsc.subcore_barrier()

    @pl.when(c == 0)
    def _():
        _sliced_copy(acc, out0_hbm, s)

    @pl.when(c == 1)
    def _():
        _sliced_copy(acc, out1_hbm, s)


BLK = 1000
GRID = N // BLK
MBLK = 5000
MGRID = N // MBLK


def _leaky(x):
    return jnp.where(x >= 0, x, 0.01 * x)


def _row_spec(w):
    return pl.BlockSpec((BLK, w), lambda i: (i, 0))


def _mrow_spec(w):
    return pl.BlockSpec((MBLK, w), lambda i: (i, 0))


def _full_spec(a, b):
    return pl.BlockSpec((a, b), lambda i: (0, 0))


def _tc_encode_body(screen, des, tweet, profile, personal, deg0, deg1,
                    Ws, bs, Wd, bd, Wt, bt, Wp, bp, Wq, bq, Wl, bl, Wg, bg,
                    x1_o, dinv_o, g0_o, g1_o, selfw_o):
    s = _leaky(jnp.dot(screen[...], Ws[...],
                       preferred_element_type=jnp.float32) + bs[...])
    d = _leaky(jnp.dot(des[...], Wd[...],
                       preferred_element_type=jnp.float32) + bd[...])
    t = _leaky(jnp.dot(tweet[...], Wt[...],
                       preferred_element_type=jnp.float32) + bt[...])
    p = _leaky(jnp.dot(profile[...], Wp[...],
                       preferred_element_type=jnp.float32) + bp[...])
    q = _leaky(jnp.dot(personal[...], Wq[...],
                       preferred_element_type=jnp.float32) + bq[...])
    x1 = jnp.concatenate([s, d, t, p, q], axis=1)
    x = _leaky(jnp.dot(x1, Wl[...], preferred_element_type=jnp.float32)
               + bl[...])
    deg = deg0[:, 0:1] + deg1[:, 0:1] + 1.0
    dinv = lax.rsqrt(deg)
    h = jnp.dot(x, Wg[...], preferred_element_type=jnp.float32)
    g = h * dinv
    x1_o[...] = x1
    dinv_o[...] = jnp.broadcast_to(dinv, (BLK, 8))
    g0_o[...] = g[:, :HALF]
    g1_o[...] = g[:, HALF:]
    selfw_o[...] = g * dinv + bg[...] + x1


_tc_encode = pl.pallas_call(
    _tc_encode_body,
    grid=(GRID,),
    in_specs=[
        _row_spec(768), _row_spec(768), _row_spec(768),
        _row_spec(5), _row_spec(7), _row_spec(8), _row_spec(8),
        _full_spec(768, 16), _full_spec(1, 16),
        _full_spec(768, 16), _full_spec(1, 16),
        _full_spec(768, 16), _full_spec(1, 16),
        _full_spec(5, 8), _full_spec(1, 8),
        _full_spec(7, 8), _full_spec(1, 8),
        _full_spec(64, 64), _full_spec(1, 64),
        _full_spec(64, 64), _full_spec(1, 64),
    ],
    out_specs=[
        _row_spec(64), _row_spec(8), _row_spec(HALF), _row_spec(HALF),
        _row_spec(64),
    ],
    out_shape=[
        jax.ShapeDtypeStruct((N, 64), jnp.float32),
        jax.ShapeDtypeStruct((N, 8), jnp.float32),
        jax.ShapeDtypeStruct((N, HALF), jnp.float32),
        jax.ShapeDtypeStruct((N, HALF), jnp.float32),
        jax.ShapeDtypeStruct((N, 64), jnp.float32),
    ],
)


def _tc_mid_body(a0, a1, dinv, selfw, x1, W, b,
                 g0_o, g1_o, selfw_o):
    di = dinv[:, 0:1]
    xl = di * jnp.concatenate([a0[...], a1[...]], axis=1) + selfw[...]
    h = jnp.dot(xl, W[...], preferred_element_type=jnp.float32)
    g = h * di
    g0_o[...] = g[:, :HALF]
    g1_o[...] = g[:, HALF:]
    selfw_o[...] = g * di + b[...] + x1[...]


_tc_mid = pl.pallas_call(
    _tc_mid_body,
    grid=(MGRID,),
    in_specs=[
        _mrow_spec(HALF), _mrow_spec(HALF), _mrow_spec(8), _mrow_spec(64),
        _mrow_spec(64),
        _full_spec(64, 64), _full_spec(1, 64),
    ],
    out_specs=[_mrow_spec(HALF), _mrow_spec(HALF), _mrow_spec(64)],
    out_shape=[
        jax.ShapeDtypeStruct((N, HALF), jnp.float32),
        jax.ShapeDtypeStruct((N, HALF), jnp.float32),
        jax.ShapeDtypeStruct((N, 64), jnp.float32),
    ],
)


def _tc_final_body(a0, a1, dinv, selfw, Wo1, bo1, Wo2, bo2, y_o):
    di = dinv[:, 0:1]
    x4 = di * jnp.concatenate([a0[...], a1[...]], axis=1) + selfw[...]
    h = _leaky(jnp.dot(x4, Wo1[...], preferred_element_type=jnp.float32)
               + bo1[...])
    z = jnp.dot(h, Wo2[...], preferred_element_type=jnp.float32) + bo2[...]
    y_o[...] = 1.0 / (1.0 + jnp.exp(-z))


_tc_final = pl.pallas_call(
    _tc_final_body,
    grid=(MGRID,),
    in_specs=[
        _mrow_spec(HALF), _mrow_spec(HALF), _mrow_spec(8), _mrow_spec(64),
        _full_spec(64, 64), _full_spec(1, 64),
        _full_spec(64, 2), _full_spec(1, 2),
    ],
    out_specs=[_mrow_spec(2)],
    out_shape=[jax.ShapeDtypeStruct((N, 2), jnp.float32)],
)


def kernel(screen, des, tweet, profile, personal, edge, edgeRelation,
           Ws_w, Ws_b, Wd_w, Wd_b, Wt_w, Wt_b, Wp_w, Wp_b, Wq_w, Wq_b,
           Wl_w, Wl_b, Wg1, bg1, Wg2, bg2, Wg3, bg3, Wo1, bo1, Wo2, bo2):
    src = edge[0].astype(jnp.int32)
    dst = edge[1].astype(jnp.int32)
    src2d = src.reshape(SEG_IDX_ROWS, SEG_CHUNK)
    dst2d = dst.reshape(SEG_IDX_ROWS, SEG_CHUNK)

    zeros8 = jnp.zeros((N, DEG_W), jnp.float32)
    ones8 = jnp.ones((DEG_CHUNK, DEG_W), jnp.float32)
    zeros32 = jnp.zeros((N, HALF), jnp.float32)

    r2 = lambda v: v.reshape(1, -1)

    deg0, deg1 = _deg_sc(dst, zeros8, ones8)

    x1, dinv, g0, g1, selfw = _tc_encode(
        screen, des, tweet, profile, personal, deg0, deg1,
        Ws_w, r2(Ws_b), Wd_w, r2(Wd_b), Wt_w, r2(Wt_b),
        Wp_w, r2(Wp_b), Wq_w, r2(Wq_b), Wl_w, r2(Wl_b), Wg1, r2(bg1))

    for W, b in ((Wg2, bg2), (Wg3, bg3)):
        a0, a1 = _seg_sc(g0, g1, src2d, dst2d, zeros32)
        g0, g1, selfw = _tc_mid(a0, a1, dinv, selfw, x1, W, r2(b))

    a0, a1 = _seg_sc(g0, g1, src2d, dst2d, zeros32)
    y, = _tc_final(a0, a1, dinv, selfw, Wo1, r2(bo1), Wo2, r2(bo2))
    return y


# speedup vs baseline: 1.0791x; 1.0200x over previous
import functools

import jax
import jax.numpy as jnp
from jax import lax
from jax.experimental import pallas as pl
from jax.experimental.pallas import tpu as pltpu
from jax.experimental.pallas import tpu_sc as plsc

N = 50000
E = 800000
NC = 2
NS = 16


DEG_W = 8
DEG_CHUNK = 5000
DEG_PER_TILE = E // (NC * NS)

_sc_mesh = plsc.VectorSubcoreMesh(
    core_axis_name="c", subcore_axis_name="s", num_cores=NC, num_subcores=NS)
_sc_params = pltpu.CompilerParams(use_tc_tiling_on_sc=False)

ROWS_A = 3128
ROWS_B = N - (NS - 1) * ROWS_A


def _sliced_copy(src_ref, dst_ref, s):
    @pl.when(s < NS - 1)
    def _():
        pltpu.sync_copy(src_ref.at[pl.ds(s * ROWS_A, ROWS_A)],
                        dst_ref.at[pl.ds(s * ROWS_A, ROWS_A)])

    @pl.when(s == NS - 1)
    def _():
        pltpu.sync_copy(src_ref.at[pl.ds((NS - 1) * ROWS_A, ROWS_B)],
                        dst_ref.at[pl.ds((NS - 1) * ROWS_A, ROWS_B)])


ZCHUNK = 1000


def _zero_fill(zsrc, acc_ref, s):
    def fill(row0, nrows):
        full = nrows // ZCHUNK
        for i in range(full):
            pltpu.sync_copy(zsrc, acc_ref.at[pl.ds(row0 + i * ZCHUNK, ZCHUNK)])
        rem = nrows - full * ZCHUNK
        if rem:
            pltpu.sync_copy(zsrc.at[pl.ds(0, rem)],
                            acc_ref.at[pl.ds(row0 + full * ZCHUNK, rem)])

    @pl.when(s < NS - 1)
    def _():
        fill(s * ROWS_A, ROWS_A)

    @pl.when(s == NS - 1)
    def _():
        fill((NS - 1) * ROWS_A, ROWS_B)


@functools.partial(
    pl.kernel,
    out_type=(
        jax.ShapeDtypeStruct((N, DEG_W), jnp.float32),
        jax.ShapeDtypeStruct((N, DEG_W), jnp.float32),
    ),
    mesh=_sc_mesh,
    compiler_params=_sc_params,
    scratch_types=[
        pltpu.VMEM((DEG_CHUNK,), jnp.int32),
        pltpu.VMEM((DEG_CHUNK, DEG_W), jnp.float32),
        pltpu.VMEM_SHARED((N, DEG_W), jnp.float32),
        pltpu.SemaphoreType.DMA,
    ],
)
def _deg_sc(edge3d_hbm, zeros8_hbm, ones8_hbm, out0_hbm, out1_hbm,
            didx, ones_v, acc, isem):
    c = lax.axis_index("c")
    s = lax.axis_index("s")
    dst2d = edge3d_hbm.at[1]
    rows_per_chunk = DEG_CHUNK // SEG_CHUNK

    pltpu.sync_copy(ones8_hbm, ones_v)
    _sliced_copy(zeros8_hbm, acc, s)
    plsc.subcore_barrier()

    base = (c * NS + s) * (DEG_PER_TILE // SEG_CHUNK)

    def body(k, carry):
        row0 = base + k * rows_per_chunk
        descs = [
            pltpu.async_copy(dst2d.at[row0 + r],
                             didx.at[pl.ds(r * SEG_CHUNK, SEG_CHUNK)], isem)
            for r in range(rows_per_chunk)
        ]
        for d in descs:
            d.wait()
        pltpu.sync_copy(ones_v, acc.at[didx], add=True)
        return carry

    lax.fori_loop(0, DEG_PER_TILE // DEG_CHUNK, body, 0)
    plsc.subcore_barrier()

    @pl.when(c == 0)
    def _():
        _sliced_copy(acc, out0_hbm, s)

    @pl.when(c == 1)
    def _():
        _sliced_copy(acc, out1_hbm, s)


SEG_CHUNK = 200
SEG_CPB = 10
SEG_NBLK = 25
SEG_IDX_ROWS = E // SEG_CHUNK
HALF = 32


@functools.partial(
    pl.kernel,
    out_type=(
        jax.ShapeDtypeStruct((N, HALF), jnp.float32),
        jax.ShapeDtypeStruct((N, HALF), jnp.float32),
    ),
    mesh=_sc_mesh,
    compiler_params=_sc_params,
    scratch_types=[
        pltpu.VMEM((SEG_CPB, SEG_CHUNK), jnp.int32),
        pltpu.VMEM((SEG_CPB, SEG_CHUNK), jnp.int32),
        pltpu.VMEM((SEG_CPB, SEG_CHUNK), jnp.int32),
        pltpu.VMEM((SEG_CPB, SEG_CHUNK), jnp.int32),
        pltpu.VMEM((SEG_CPB, SEG_CHUNK), jnp.int32),
        pltpu.VMEM((SEG_CPB, SEG_CHUNK), jnp.int32),
        pltpu.VMEM((SEG_CHUNK, HALF), jnp.float32),
        pltpu.VMEM((SEG_CHUNK, HALF), jnp.float32),
        pltpu.VMEM_SHARED((N, HALF), jnp.float32),
        pltpu.SemaphoreType.DMA,
        pltpu.SemaphoreType.DMA,
        pltpu.SemaphoreType.DMA,
        pltpu.SemaphoreType.DMA,
        pltpu.SemaphoreType.DMA,
        pltpu.SemaphoreType.DMA,
        pltpu.SemaphoreType.DMA,
    ],
)
def _seg_sc(g0_hbm, g1_hbm, edge3d_hbm, zeros32_hbm,
            out0_hbm, out1_hbm,
            sblk0, dblk0, sblk1, dblk1, sblk2, dblk2, rows0, rows1, acc,
            gsem0, gsem1, ssem0, ssem1, isem0, isem1, isem2):
    c = lax.axis_index("c")
    s = lax.axis_index("s")

    src2d_hbm = edge3d_hbm.at[0]
    dst2d_hbm = edge3d_hbm.at[1]

    _sliced_copy(zeros32_hbm, acc, s)
    plsc.subcore_barrier()

    rowsb = (rows0, rows1)
    gsem = (gsem0, gsem1)
    ssem = (ssem0, ssem1)
    idxb = ((sblk0, dblk0, isem0), (sblk1, dblk1, isem1),
            (sblk2, dblk2, isem2))
    tile_row0 = s * (SEG_NBLK * SEG_CPB)

    def load_idx(blk, bufs):
        sb, db, sem = bufs
        base = tile_row0 + blk * SEG_CPB
        d1 = pltpu.async_copy(src2d_hbm.at[pl.ds(base, SEG_CPB)], sb, sem)
        d2 = pltpu.async_copy(dst2d_hbm.at[pl.ds(base, SEG_CPB)], db, sem)
        return d1, d2

    def wait_idx(bufs):
        sb, db, sem = bufs
        pltpu.make_async_copy(
            src2d_hbm.at[pl.ds(0, SEG_CPB)], sb, sem).wait()
        pltpu.make_async_copy(
            dst2d_hbm.at[pl.ds(0, SEG_CPB)], db, sem).wait()

    def wait_prev_scatter(b, db):
        pltpu.make_async_copy(rowsb[b], acc.at[db.at[0]], ssem[b]).wait()

    def process_block(g_ref, bufs, first, prefetch):
        sb, db, sem = bufs
        g_descs = [None] * SEG_CPB
        s_descs = [None] * SEG_CPB
        for j in range(SEG_CPB):
            b = j & 1
            if j >= 2:
                s_descs[j - 2].wait()
            elif not first:
                wait_prev_scatter(b, db)
            g_descs[j] = pltpu.async_copy(g_ref.at[sb.at[j]], rowsb[b],
                                          gsem[b])
            if j >= 1:
                g_descs[j - 1].wait()
                s_descs[j - 1] = pltpu.async_copy(
                    rowsb[(j - 1) & 1], acc.at[db.at[j - 1]],
                    ssem[(j - 1) & 1], add=True)
        last = SEG_CPB - 1
        g_descs[last].wait()
        s_descs[last] = pltpu.async_copy(
            rowsb[last & 1], acc.at[db.at[last]], ssem[last & 1], add=True)
        if prefetch is not None:
            prefetch()

    def run(g_ref):
        d0 = load_idx(0, idxb[0])
        load_idx(1, idxb[1])
        load_idx(2, idxb[2])
        d0[0].wait()
        d0[1].wait()
        process_block(g_ref, idxb[0], True, None)

        def body(j2, carry):
            for k, off in ((1, 1), (2, 2), (0, 3)):
                blk = 3 * j2 + off
                bufs = idxb[k]
                wait_idx(bufs)

                def prefetch(blk=blk, k=k):
                    @pl.when(blk + 2 <= SEG_NBLK - 1)
                    def _():
                        load_idx(blk + 2, idxb[(k + 2) % 3])

                process_block(g_ref, bufs, False, prefetch)
            return carry

        lax.fori_loop(0, (SEG_NBLK - 1) // 3, body, 0)
        wait_prev_scatter(0, dblk0)
        wait_prev_scatter(1, dblk0)

    @pl.when(c == 0)
    def _():
        run(g0_hbm)

    @pl.when(c == 1)
    def _():
        run(g1_hbm)

    plsc.subcore_barrier()

    @pl.when(c == 0)
    def _():
        _sliced_copy(acc, out0_hbm, s)

    @pl.when(c == 1)
    def _():
        _sliced_copy(acc, out1_hbm, s)


BLK = 1000
GRID = N // BLK
MBLK = 5000
MGRID = N // MBLK


def _leaky(x):
    return jnp.where(x >= 0, x, 0.01 * x)


def _row_spec(w):
    return pl.BlockSpec((BLK, w), lambda i: (i, 0))


def _mrow_spec(w):
    return pl.BlockSpec((MBLK, w), lambda i: (i, 0))


def _full_spec(a, b):
    return pl.BlockSpec((a, b), lambda i: (0, 0))


def _tc_encode_body(screen, des, tweet, profile, personal, deg0, deg1,
                    Ws, bs, Wd, bd, Wt, bt, Wp, bp, Wq, bq, Wl, bl, Wg, bg,
                    x1_o, dinv_o, g0_o, g1_o, selfw_o):
    s = _leaky(jnp.dot(screen[...], Ws[...],
                       preferred_element_type=jnp.float32) + bs[...])
    d = _leaky(jnp.dot(des[...], Wd[...],
                       preferred_element_type=jnp.float32) + bd[...])
    t = _leaky(jnp.dot(tweet[...], Wt[...],
                       preferred_element_type=jnp.float32) + bt[...])
    p = _leaky(jnp.dot(profile[...], Wp[...],
                       preferred_element_type=jnp.float32) + bp[...])
    q = _leaky(jnp.dot(personal[...], Wq[...],
                       preferred_element_type=jnp.float32) + bq[...])
    x1 = jnp.concatenate([s, d, t, p, q], axis=1)
    x = _leaky(jnp.dot(x1, Wl[...], preferred_element_type=jnp.float32)
               + bl[...])
    deg = deg0[:, 0:1] + deg1[:, 0:1] + 1.0
    dinv = lax.rsqrt(deg)
    h = jnp.dot(x, Wg[...], preferred_element_type=jnp.float32)
    g = h * dinv
    x1_o[...] = x1
    dinv_o[...] = jnp.broadcast_to(dinv, (BLK, 8))
    g0_o[...] = g[:, :HALF]
    g1_o[...] = g[:, HALF:]
    selfw_o[...] = g * dinv + bg[...] + x1


_tc_encode = pl.pallas_call(
    _tc_encode_body,
    grid=(GRID,),
    in_specs=[
        _row_spec(768), _row_spec(768), _row_spec(768),
        _row_spec(5), _row_spec(7), _row_spec(8), _row_spec(8),
        _full_spec(768, 16), _full_spec(1, 16),
        _full_spec(768, 16), _full_spec(1, 16),
        _full_spec(768, 16), _full_spec(1, 16),
        _full_spec(5, 8), _full_spec(1, 8),
        _full_spec(7, 8), _full_spec(1, 8),
        _full_spec(64, 64), _full_spec(1, 64),
        _full_spec(64, 64), _full_spec(1, 64),
    ],
    out_specs=[
        _row_spec(64), _row_spec(8), _row_spec(HALF), _row_spec(HALF),
        _row_spec(64),
    ],
    out_shape=[
        jax.ShapeDtypeStruct((N, 64), jnp.float32),
        jax.ShapeDtypeStruct((N, 8), jnp.float32),
        jax.ShapeDtypeStruct((N, HALF), jnp.float32),
        jax.ShapeDtypeStruct((N, HALF), jnp.float32),
        jax.ShapeDtypeStruct((N, 64), jnp.float32),
    ],
)


def _tc_mid_body(a0, a1, dinv, selfw, x1, W, b,
                 g0_o, g1_o, selfw_o):
    di = dinv[:, 0:1]
    xl = di * jnp.concatenate([a0[...], a1[...]], axis=1) + selfw[...]
    h = jnp.dot(xl, W[...], preferred_element_type=jnp.float32)
    g = h * di
    g0_o[...] = g[:, :HALF]
    g1_o[...] = g[:, HALF:]
    selfw_o[...] = g * di + b[...] + x1[...]


_tc_mid = pl.pallas_call(
    _tc_mid_body,
    grid=(MGRID,),
    in_specs=[
        _mrow_spec(HALF), _mrow_spec(HALF), _mrow_spec(8), _mrow_spec(64),
        _mrow_spec(64),
        _full_spec(64, 64), _full_spec(1, 64),
    ],
    out_specs=[_mrow_spec(HALF), _mrow_spec(HALF), _mrow_spec(64)],
    out_shape=[
        jax.ShapeDtypeStruct((N, HALF), jnp.float32),
        jax.ShapeDtypeStruct((N, HALF), jnp.float32),
        jax.ShapeDtypeStruct((N, 64), jnp.float32),
    ],
)


def _tc_final_body(a0, a1, dinv, selfw, Wo1, bo1, Wo2, bo2, y_o):
    di = dinv[:, 0:1]
    x4 = di * jnp.concatenate([a0[...], a1[...]], axis=1) + selfw[...]
    h = _leaky(jnp.dot(x4, Wo1[...], preferred_element_type=jnp.float32)
               + bo1[...])
    z = jnp.dot(h, Wo2[...], preferred_element_type=jnp.float32) + bo2[...]
    y_o[...] = 1.0 / (1.0 + jnp.exp(-z))


_tc_final = pl.pallas_call(
    _tc_final_body,
    grid=(MGRID,),
    in_specs=[
        _mrow_spec(HALF), _mrow_spec(HALF), _mrow_spec(8), _mrow_spec(64),
        _full_spec(64, 64), _full_spec(1, 64),
        _full_spec(64, 2), _full_spec(1, 2),
    ],
    out_specs=[_mrow_spec(2)],
    out_shape=[jax.ShapeDtypeStruct((N, 2), jnp.float32)],
)


def kernel(screen, des, tweet, profile, personal, edge, edgeRelation,
           Ws_w, Ws_b, Wd_w, Wd_b, Wt_w, Wt_b, Wp_w, Wp_b, Wq_w, Wq_b,
           Wl_w, Wl_b, Wg1, bg1, Wg2, bg2, Wg3, bg3, Wo1, bo1, Wo2, bo2):
    edge3d = edge.astype(jnp.int32).reshape(2, SEG_IDX_ROWS, SEG_CHUNK)

    zeros8 = jnp.zeros((N, DEG_W), jnp.float32)
    ones8 = jnp.ones((DEG_CHUNK, DEG_W), jnp.float32)
    zeros32 = jnp.zeros((N, HALF), jnp.float32)

    r2 = lambda v: v.reshape(1, -1)

    deg0, deg1 = _deg_sc(edge3d, zeros8, ones8)

    x1, dinv, g0, g1, selfw = _tc_encode(
        screen, des, tweet, profile, personal, deg0, deg1,
        Ws_w, r2(Ws_b), Wd_w, r2(Wd_b), Wt_w, r2(Wt_b),
        Wp_w, r2(Wp_b), Wq_w, r2(Wq_b), Wl_w, r2(Wl_b), Wg1, r2(bg1))

    for W, b in ((Wg2, bg2), (Wg3, bg3)):
        a0, a1 = _seg_sc(g0, g1, edge3d, zeros32)
        g0, g1, selfw = _tc_mid(a0, a1, dinv, selfw, x1, W, r2(b))

    a0, a1 = _seg_sc(g0, g1, edge3d, zeros32)
    y, = _tc_final(a0, a1, dinv, selfw, Wo1, r2(bo1), Wo2, r2(bo2))
    return y


# speedup vs baseline: 1.0897x; 1.0098x over previous
import functools

import jax
import jax.numpy as jnp
from jax import lax
from jax.experimental import pallas as pl
from jax.experimental.pallas import tpu as pltpu
from jax.experimental.pallas import tpu_sc as plsc

N = 50000
E = 800000
NC = 2
NS = 16


DEG_W = 8
DEG_CHUNK = 5000
DEG_PER_TILE = E // (NC * NS)

_sc_mesh = plsc.VectorSubcoreMesh(
    core_axis_name="c", subcore_axis_name="s", num_cores=NC, num_subcores=NS)
_sc_params = pltpu.CompilerParams(use_tc_tiling_on_sc=False)

ROWS_A = 3128
ROWS_B = N - (NS - 1) * ROWS_A


def _sliced_copy(src_ref, dst_ref, s):
    @pl.when(s < NS - 1)
    def _():
        pltpu.sync_copy(src_ref.at[pl.ds(s * ROWS_A, ROWS_A)],
                        dst_ref.at[pl.ds(s * ROWS_A, ROWS_A)])

    @pl.when(s == NS - 1)
    def _():
        pltpu.sync_copy(src_ref.at[pl.ds((NS - 1) * ROWS_A, ROWS_B)],
                        dst_ref.at[pl.ds((NS - 1) * ROWS_A, ROWS_B)])


ZCHUNK = 1000


def _zero_fill(zsrc, acc_ref, s):
    def fill(row0, nrows):
        full = nrows // ZCHUNK
        for i in range(full):
            pltpu.sync_copy(zsrc, acc_ref.at[pl.ds(row0 + i * ZCHUNK, ZCHUNK)])
        rem = nrows - full * ZCHUNK
        if rem:
            pltpu.sync_copy(zsrc.at[pl.ds(0, rem)],
                            acc_ref.at[pl.ds(row0 + full * ZCHUNK, rem)])

    @pl.when(s < NS - 1)
    def _():
        fill(s * ROWS_A, ROWS_A)

    @pl.when(s == NS - 1)
    def _():
        fill((NS - 1) * ROWS_A, ROWS_B)


@functools.partial(
    pl.kernel,
    out_type=jax.ShapeDtypeStruct((2, N, DEG_W), jnp.float32),
    mesh=_sc_mesh,
    compiler_params=_sc_params,
    scratch_types=[
        pltpu.VMEM((DEG_CHUNK,), jnp.int32),
        pltpu.VMEM((DEG_CHUNK, DEG_W), jnp.float32),
        pltpu.VMEM_SHARED((N, DEG_W), jnp.float32),
        pltpu.SemaphoreType.DMA,
    ],
)
def _deg_sc(edge3d_hbm, zeros8_hbm, ones8_hbm, out_hbm,
            didx, ones_v, acc, isem):
    c = lax.axis_index("c")
    s = lax.axis_index("s")
    dst2d = edge3d_hbm.at[1]
    rows_per_chunk = DEG_CHUNK // SEG_CHUNK

    pltpu.sync_copy(ones8_hbm, ones_v)
    _sliced_copy(zeros8_hbm, acc, s)
    plsc.subcore_barrier()

    base = (c * NS + s) * (DEG_PER_TILE // SEG_CHUNK)

    def body(k, carry):
        row0 = base + k * rows_per_chunk
        descs = [
            pltpu.async_copy(dst2d.at[row0 + r],
                             didx.at[pl.ds(r * SEG_CHUNK, SEG_CHUNK)], isem)
            for r in range(rows_per_chunk)
        ]
        for d in descs:
            d.wait()
        pltpu.sync_copy(ones_v, acc.at[didx], add=True)
        return carry

    lax.fori_loop(0, DEG_PER_TILE // DEG_CHUNK, body, 0)
    plsc.subcore_barrier()

    @pl.when(c == 0)
    def _():
        _sliced_copy(acc, out_hbm.at[0], s)

    @pl.when(c == 1)
    def _():
        _sliced_copy(acc, out_hbm.at[1], s)


SEG_CHUNK = 200
SEG_CPB = 10
SEG_NBLK = 25
SEG_IDX_ROWS = E // SEG_CHUNK
HALF = 32


@functools.partial(
    pl.kernel,
    out_type=jax.ShapeDtypeStruct((2, N, HALF), jnp.float32),
    mesh=_sc_mesh,
    compiler_params=_sc_params,
    scratch_types=[
        pltpu.VMEM((SEG_CPB, SEG_CHUNK), jnp.int32),
        pltpu.VMEM((SEG_CPB, SEG_CHUNK), jnp.int32),
        pltpu.VMEM((SEG_CPB, SEG_CHUNK), jnp.int32),
        pltpu.VMEM((SEG_CPB, SEG_CHUNK), jnp.int32),
        pltpu.VMEM((SEG_CPB, SEG_CHUNK), jnp.int32),
        pltpu.VMEM((SEG_CPB, SEG_CHUNK), jnp.int32),
        pltpu.VMEM((SEG_CHUNK, HALF), jnp.float32),
        pltpu.VMEM((SEG_CHUNK, HALF), jnp.float32),
        pltpu.VMEM_SHARED((N, HALF), jnp.float32),
        pltpu.SemaphoreType.DMA,
        pltpu.SemaphoreType.DMA,
        pltpu.SemaphoreType.DMA,
        pltpu.SemaphoreType.DMA,
        pltpu.SemaphoreType.DMA,
        pltpu.SemaphoreType.DMA,
        pltpu.SemaphoreType.DMA,
    ],
)
def _seg_sc(g_hbm, edge3d_hbm, zeros32_hbm, out_hbm,
            sblk0, dblk0, sblk1, dblk1, sblk2, dblk2, rows0, rows1, acc,
            gsem0, gsem1, ssem0, ssem1, isem0, isem1, isem2):
    c = lax.axis_index("c")
    s = lax.axis_index("s")

    src2d_hbm = edge3d_hbm.at[0]
    dst2d_hbm = edge3d_hbm.at[1]

    _sliced_copy(zeros32_hbm, acc, s)
    plsc.subcore_barrier()

    rowsb = (rows0, rows1)
    gsem = (gsem0, gsem1)
    ssem = (ssem0, ssem1)
    idxb = ((sblk0, dblk0, isem0), (sblk1, dblk1, isem1),
            (sblk2, dblk2, isem2))
    tile_row0 = s * (SEG_NBLK * SEG_CPB)

    def load_idx(blk, bufs):
        sb, db, sem = bufs
        base = tile_row0 + blk * SEG_CPB
        d1 = pltpu.async_copy(src2d_hbm.at[pl.ds(base, SEG_CPB)], sb, sem)
        d2 = pltpu.async_copy(dst2d_hbm.at[pl.ds(base, SEG_CPB)], db, sem)
        return d1, d2

    def wait_idx(bufs):
        sb, db, sem = bufs
        pltpu.make_async_copy(
            src2d_hbm.at[pl.ds(0, SEG_CPB)], sb, sem).wait()
        pltpu.make_async_copy(
            dst2d_hbm.at[pl.ds(0, SEG_CPB)], db, sem).wait()

    def wait_prev_scatter(b, db):
        pltpu.make_async_copy(rowsb[b], acc.at[db.at[0]], ssem[b]).wait()

    def process_block(g_ref, bufs, first, prefetch):
        sb, db, sem = bufs
        g_descs = [None] * SEG_CPB
        s_descs = [None] * SEG_CPB
        for j in range(SEG_CPB):
            b = j & 1
            if j >= 2:
                s_descs[j - 2].wait()
            elif not first:
                wait_prev_scatter(b, db)
            g_descs[j] = pltpu.async_copy(g_ref.at[sb.at[j]], rowsb[b],
                                          gsem[b])
            if j >= 1:
                g_descs[j - 1].wait()
                s_descs[j - 1] = pltpu.async_copy(
                    rowsb[(j - 1) & 1], acc.at[db.at[j - 1]],
                    ssem[(j - 1) & 1], add=True)
        last = SEG_CPB - 1
        g_descs[last].wait()
        s_descs[last] = pltpu.async_copy(
            rowsb[last & 1], acc.at[db.at[last]], ssem[last & 1], add=True)
        if prefetch is not None:
            prefetch()

    def run(g_ref):
        d0 = load_idx(0, idxb[0])
        load_idx(1, idxb[1])
        load_idx(2, idxb[2])
        d0[0].wait()
        d0[1].wait()
        process_block(g_ref, idxb[0], True, None)

        def body(j2, carry):
            for k, off in ((1, 1), (2, 2), (0, 3)):
                blk = 3 * j2 + off
                bufs = idxb[k]
                wait_idx(bufs)

                def prefetch(blk=blk, k=k):
                    @pl.when(blk + 2 <= SEG_NBLK - 1)
                    def _():
                        load_idx(blk + 2, idxb[(k + 2) % 3])

                process_block(g_ref, bufs, False, prefetch)
            return carry

        lax.fori_loop(0, (SEG_NBLK - 1) // 3, body, 0)
        wait_prev_scatter(0, dblk0)
        wait_prev_scatter(1, dblk0)

    @pl.when(c == 0)
    def _():
        run(g_hbm.at[0])

    @pl.when(c == 1)
    def _():
        run(g_hbm.at[1])

    plsc.subcore_barrier()

    @pl.when(c == 0)
    def _():
        _sliced_copy(acc, out_hbm.at[0], s)

    @pl.when(c == 1)
    def _():
        _sliced_copy(acc, out_hbm.at[1], s)


BLK = 1000
GRID = N // BLK
MBLK = 5000
MGRID = N // MBLK


def _leaky(x):
    return jnp.where(x >= 0, x, 0.01 * x)


def _row_spec(w):
    return pl.BlockSpec((BLK, w), lambda i: (i, 0))


def _mrow_spec(w):
    return pl.BlockSpec((MBLK, w), lambda i: (i, 0))


def _full_spec(a, b):
    return pl.BlockSpec((a, b), lambda i: (0, 0))


def _tc_encode_body(screen, des, tweet, profile, personal, degc,
                    Ws, bs, Wd, bd, Wt, bt, Wp, bp, Wq, bq, Wl, bl, Wg, bg,
                    x1_o, dinv_o, g_o, selfw_o):
    s = _leaky(jnp.dot(screen[...], Ws[...],
                       preferred_element_type=jnp.float32) + bs[...])
    d = _leaky(jnp.dot(des[...], Wd[...],
                       preferred_element_type=jnp.float32) + bd[...])
    t = _leaky(jnp.dot(tweet[...], Wt[...],
                       preferred_element_type=jnp.float32) + bt[...])
    p = _leaky(jnp.dot(profile[...], Wp[...],
                       preferred_element_type=jnp.float32) + bp[...])
    q = _leaky(jnp.dot(personal[...], Wq[...],
                       preferred_element_type=jnp.float32) + bq[...])
    x1 = jnp.concatenate([s, d, t, p, q], axis=1)
    x = _leaky(jnp.dot(x1, Wl[...], preferred_element_type=jnp.float32)
               + bl[...])
    deg = degc[0, :, 0:1] + degc[1, :, 0:1] + 1.0
    dinv = lax.rsqrt(deg)
    h = jnp.dot(x, Wg[...], preferred_element_type=jnp.float32)
    g = h * dinv
    x1_o[...] = x1
    dinv_o[...] = jnp.broadcast_to(dinv, (BLK, 8))
    g_o[0] = g[:, :HALF]
    g_o[1] = g[:, HALF:]
    selfw_o[...] = g * dinv + bg[...] + x1


_tc_encode = pl.pallas_call(
    _tc_encode_body,
    grid=(GRID,),
    in_specs=[
        _row_spec(768), _row_spec(768), _row_spec(768),
        _row_spec(5), _row_spec(7),
        pl.BlockSpec((2, BLK, 8), lambda i: (0, i, 0)),
        _full_spec(768, 16), _full_spec(1, 16),
        _full_spec(768, 16), _full_spec(1, 16),
        _full_spec(768, 16), _full_spec(1, 16),
        _full_spec(5, 8), _full_spec(1, 8),
        _full_spec(7, 8), _full_spec(1, 8),
        _full_spec(64, 64), _full_spec(1, 64),
        _full_spec(64, 64), _full_spec(1, 64),
    ],
    out_specs=[
        _row_spec(64), _row_spec(8),
        pl.BlockSpec((2, BLK, HALF), lambda i: (0, i, 0)),
        _row_spec(64),
    ],
    out_shape=[
        jax.ShapeDtypeStruct((N, 64), jnp.float32),
        jax.ShapeDtypeStruct((N, 8), jnp.float32),
        jax.ShapeDtypeStruct((2, N, HALF), jnp.float32),
        jax.ShapeDtypeStruct((N, 64), jnp.float32),
    ],
)


def _tc_mid_body(ac, dinv, selfw, x1, W, b,
                 g_o, selfw_o):
    di = dinv[:, 0:1]
    xl = di * jnp.concatenate([ac[0], ac[1]], axis=1) + selfw[...]
    h = jnp.dot(xl, W[...], preferred_element_type=jnp.float32)
    g = h * di
    g_o[0] = g[:, :HALF]
    g_o[1] = g[:, HALF:]
    selfw_o[...] = g * di + b[...] + x1[...]


_tc_mid = pl.pallas_call(
    _tc_mid_body,
    grid=(MGRID,),
    in_specs=[
        pl.BlockSpec((2, MBLK, HALF), lambda i: (0, i, 0)),
        _mrow_spec(8), _mrow_spec(64), _mrow_spec(64),
        _full_spec(64, 64), _full_spec(1, 64),
    ],
    out_specs=[
        pl.BlockSpec((2, MBLK, HALF), lambda i: (0, i, 0)),
        _mrow_spec(64),
    ],
    out_shape=[
        jax.ShapeDtypeStruct((2, N, HALF), jnp.float32),
        jax.ShapeDtypeStruct((N, 64), jnp.float32),
    ],
)


def _tc_final_body(ac, dinv, selfw, Wo1, bo1, Wo2, bo2, y_o):
    di = dinv[:, 0:1]
    x4 = di * jnp.concatenate([ac[0], ac[1]], axis=1) + selfw[...]
    h = _leaky(jnp.dot(x4, Wo1[...], preferred_element_type=jnp.float32)
               + bo1[...])
    z = jnp.dot(h, Wo2[...], preferred_element_type=jnp.float32) + bo2[...]
    y_o[...] = 1.0 / (1.0 + jnp.exp(-z))


_tc_final = pl.pallas_call(
    _tc_final_body,
    grid=(MGRID,),
    in_specs=[
        pl.BlockSpec((2, MBLK, HALF), lambda i: (0, i, 0)),
        _mrow_spec(8), _mrow_spec(64),
        _full_spec(64, 64), _full_spec(1, 64),
        _full_spec(64, 2), _full_spec(1, 2),
    ],
    out_specs=[_mrow_spec(2)],
    out_shape=[jax.ShapeDtypeStruct((N, 2), jnp.float32)],
)


def kernel(screen, des, tweet, profile, personal, edge, edgeRelation,
           Ws_w, Ws_b, Wd_w, Wd_b, Wt_w, Wt_b, Wp_w, Wp_b, Wq_w, Wq_b,
           Wl_w, Wl_b, Wg1, bg1, Wg2, bg2, Wg3, bg3, Wo1, bo1, Wo2, bo2):
    edge3d = edge.astype(jnp.int32).reshape(2, SEG_IDX_ROWS, SEG_CHUNK)

    zeros8 = jnp.zeros((N, DEG_W), jnp.float32)
    ones8 = jnp.ones((DEG_CHUNK, DEG_W), jnp.float32)
    zeros32 = jnp.zeros((N, HALF), jnp.float32)

    r2 = lambda v: v.reshape(1, -1)

    degc = _deg_sc(edge3d, zeros8, ones8)

    x1, dinv, g, selfw = _tc_encode(
        screen, des, tweet, profile, personal, degc,
        Ws_w, r2(Ws_b), Wd_w, r2(Wd_b), Wt_w, r2(Wt_b),
        Wp_w, r2(Wp_b), Wq_w, r2(Wq_b), Wl_w, r2(Wl_b), Wg1, r2(bg1))

    for W, b in ((Wg2, bg2), (Wg3, bg3)):
        a = _seg_sc(g, edge3d, zeros32)
        g, selfw = _tc_mid(a, dinv, selfw, x1, W, r2(b))

    a = _seg_sc(g, edge3d, zeros32)
    y, = _tc_final(a, dinv, selfw, Wo1, r2(bo1), Wo2, r2(bo2))
    return y
